# baseline (speedup 1.0000x reference)
import functools
import numpy as np
import jax
import jax.numpy as jnp
from jax import lax
from jax.experimental import pallas as pl
from jax.experimental.pallas import tpu as pltpu
from jax.experimental.pallas import tpu_sc as plsc

_MODEL_DIM = 64
_MAX_POS = 512


def _positional_encoding(position, model_dim):
    pos = np.arange(position)[:, np.newaxis].astype(np.float32)
    i = np.arange(model_dim)[np.newaxis, :].astype(np.float32)
    angle_rates = 1.0 / np.power(10000, 2 * (i // 2) / np.float32(model_dim))
    angle_rads = pos * angle_rates
    angle_rads[:, 0::2] = np.sin(angle_rads[:, 0::2])
    angle_rads[:, 1::2] = np.cos(angle_rads[:, 1::2])
    return angle_rads.astype(np.float32)


_POS_ENC = _positional_encoding(_MAX_POS, _MODEL_DIM)


@functools.partial(jax.jit, static_argnums=(3, 4, 5))
def _embed(idx_flat, table, pos, batch, seq, dim):
    B = batch * seq
    NC, NS = 2, 16
    NW = NC * NS
    rows_per_w = B // NW
    chunk = 256
    n_chunks = rows_per_w // chunk
    n_groups = chunk // 16
    nvec = dim // 16
    scale = float(np.sqrt(dim))

    mesh = plsc.VectorSubcoreMesh(core_axis_name="c", subcore_axis_name="s")

    @functools.partial(
        pl.kernel,
        mesh=mesh,
        compiler_params=pltpu.CompilerParams(needs_layout_passes=False),
        out_type=jax.ShapeDtypeStruct((B, dim), jnp.float32),
        scratch_types=[
            pltpu.VMEM((rows_per_w,), jnp.int32),
            pltpu.VMEM((2, chunk, dim), jnp.float32),
            pltpu.VMEM((seq, dim), jnp.float32),
            pltpu.SemaphoreType.DMA,
            pltpu.SemaphoreType.DMA,
            pltpu.SemaphoreType.DMA,
            pltpu.SemaphoreType.DMA,
        ],
    )
    def k(idx_hbm, table_hbm, pos_hbm, out_hbm,
          idx_v, rows_v, pos_v, gsem0, gsem1, wsem0, wsem1):
        wid = lax.axis_index("s") * NC + lax.axis_index("c")
        base = wid * rows_per_w
        pltpu.sync_copy(idx_hbm.at[pl.ds(base, rows_per_w)], idx_v)
        pltpu.sync_copy(pos_hbm, pos_v)
        gsems = (gsem0, gsem1)
        wsems = (wsem0, wsem1)

        def fire(c, p):
            buf = rows_v.at[p]

            def fire_group(g, carry2):
                v = idx_v[pl.ds(c * chunk + g * 16, 16)]
                for l in range(16):
                    pltpu.make_async_copy(
                        table_hbm.at[pl.ds(v[l], 1)],
                        buf.at[pl.ds(g * 16 + l, 1)],
                        gsems[p],
                    ).start()
                return carry2

            lax.fori_loop(0, n_groups, fire_group, 0)

        def drain_gather(p):
            pltpu.make_async_copy(
                table_hbm.at[pl.ds(0, chunk)], rows_v.at[p], gsems[p]
            ).wait()

        def drain_write(p):
            pltpu.make_async_copy(
                table_hbm.at[pl.ds(0, chunk)], rows_v.at[p], wsems[p]
            ).wait()

        def compute_and_write(c, p):
            buf = rows_v.at[p]
            start = lax.rem(c * chunk, seq)

            def row_body(i, carry2):
                for r in range(2):
                    k = 2 * i + r
                    prow = lax.rem(start + k, seq)
                    for j in range(nvec):
                        sl = pl.ds(j * 16, 16)
                        buf[k, sl] = buf[k, sl] * scale + pos_v[prow, sl]
                return carry2

            lax.fori_loop(0, chunk // 2, row_body, 0)
            pltpu.make_async_copy(
                buf, out_hbm.at[pl.ds(base + c * chunk, chunk)], wsems[p]
            ).start()

        def steady(c, p):
            @pl.when(c + 1 < n_chunks)
            def _():
                @pl.when(c >= 1)
                def __():
                    drain_write(1 - p)

                fire(c + 1, 1 - p)

            drain_gather(p)
            compute_and_write(c, p)

        fire(0, 0)

        def chunk_pair(h, carry):
            c = h * 2
            steady(c, 0)
            steady(c + 1, 1)
            return carry

        lax.fori_loop(0, n_chunks // 2, chunk_pair, 0)
        drain_write(0)
        drain_write(1)

    return k(idx_flat, table, pos)


def kernel(inp, table, training):
    batch, seq = inp.shape
    dim = table.shape[1]
    pos = jnp.asarray(_POS_ENC[:seq])
    out = _embed(inp.reshape(-1), table, pos, batch, seq, dim)
    return out.reshape(batch, seq, dim)
